# baseline (speedup 1.0000x reference)
import functools

import jax
import jax.numpy as jnp
from jax.experimental import pallas as pl
from jax.experimental.pallas import tpu as pltpu


def _tile_n(n, cap):
    for t in (20480, 16384, 10240, 8192, 5120, 4096, 2048, 1024, 512, 256,
              128, 64, 32, 16, 8):
        if t <= cap and n % t == 0:
            return t
    return n


_CW = 256


def _pool_row(h, w_ref, b_ref):
    cout = w_ref.shape[1]
    rms = []
    for c in range(0, cout, _CW):
        z = _dot(h, w_ref[:, c:c + _CW])
        rms.append(jnp.max(z, axis=0, keepdims=True))
    return jnp.concatenate(rms, axis=1) + b_ref[...]


def _row(v):
    return v.reshape(1, -1)


def _dot(a, b):
    return jnp.dot(a, b, preferred_element_type=jnp.float32)


def _dot_ta(a, b):
    return jax.lax.dot_general(a, b, (((0,), (0,)), ((), ())),
                               preferred_element_type=jnp.float32)


def _mlp_pool_body(x_ref, *args):
    pool_ref = args[-1]

    @pl.when(pl.program_id(1) == 0)
    def _():
        pool_ref[...] = jnp.full(pool_ref.shape, -jnp.inf, jnp.float32)

    h = jnp.maximum(_dot_ta(x_ref[0], args[0][...]) + args[1][...], 0.0)
    for i in range(2, len(args) - 3, 2):
        h = jnp.maximum(_dot(h, args[i][...]) + args[i + 1][...], 0.0)
    rm = _pool_row(h, args[-3], args[-2])
    pool_ref[0] = jnp.maximum(pool_ref[0], jnp.maximum(rm, 0.0))


def _mlp_pool(xp, tn, wbs):
    B, cin, N = xp.shape
    cout = wbs[-2].shape[1]
    specs = [pl.BlockSpec((1, cin, tn), lambda b, n: (b, 0, n))]
    ops = []
    for w, b in zip(wbs[::2], wbs[1::2]):
        specs.append(pl.BlockSpec(w.shape, lambda b, n: (0, 0)))
        specs.append(pl.BlockSpec((1, w.shape[1]), lambda b, n: (0, 0)))
        ops += [w, _row(b)]
    out = pl.pallas_call(
        _mlp_pool_body,
        out_shape=jax.ShapeDtypeStruct((B, 1, cout), jnp.float32),
        grid=(B, N // tn),
        in_specs=specs,
        out_specs=pl.BlockSpec((1, 1, cout), lambda b, n: (b, 0, 0)),
        compiler_params=pltpu.CompilerParams(
            dimension_semantics=("parallel", "arbitrary")),
    )(xp, *ops)
    return out.reshape(B, cout)


def _backbone_body(x_ref, w1_ref, b1_ref, w2_ref, b2_ref,
                   u1_ref, c1_ref, u2_ref, c2_ref, u3_ref, c3_ref,
                   h3_ref, pool_ref):
    @pl.when(pl.program_id(1) == 0)
    def _():
        pool_ref[...] = jnp.full(pool_ref.shape, -jnp.inf, jnp.float32)

    h = jnp.maximum(_dot_ta(x_ref[0], w1_ref[0]) + b1_ref[...], 0.0)
    h3 = jnp.maximum(_dot(h, w2_ref[...]) + b2_ref[...], 0.0)
    h3_ref[0] = h3
    t = jnp.maximum(_dot(h3, u1_ref[...]) + c1_ref[...], 0.0)
    t = jnp.maximum(_dot(t, u2_ref[...]) + c2_ref[...], 0.0)
    rm = _pool_row(t, u3_ref, c3_ref)
    pool_ref[0] = jnp.maximum(pool_ref[0], jnp.maximum(rm, 0.0))


def _backbone(xp, tn, w1p, b1, w2, b2, u1, c1, u2, c2, u3, c3):
    B, cin, N = xp.shape
    c64 = w2.shape[1]
    cg = u3.shape[1]
    h3, pool = pl.pallas_call(
        _backbone_body,
        out_shape=(jax.ShapeDtypeStruct((B, N, c64), jnp.float32),
                   jax.ShapeDtypeStruct((B, 1, cg), jnp.float32)),
        grid=(B, N // tn),
        in_specs=[
            pl.BlockSpec((1, cin, tn), lambda b, n: (b, 0, n)),
            pl.BlockSpec((1, cin, w1p.shape[2]), lambda b, n: (b, 0, 0)),
            pl.BlockSpec((1, w1p.shape[2]), lambda b, n: (0, 0)),
            pl.BlockSpec(w2.shape, lambda b, n: (0, 0)),
            pl.BlockSpec((1, c64), lambda b, n: (0, 0)),
            pl.BlockSpec(u1.shape, lambda b, n: (0, 0)),
            pl.BlockSpec((1, u1.shape[1]), lambda b, n: (0, 0)),
            pl.BlockSpec(u2.shape, lambda b, n: (0, 0)),
            pl.BlockSpec((1, u2.shape[1]), lambda b, n: (0, 0)),
            pl.BlockSpec(u3.shape, lambda b, n: (0, 0)),
            pl.BlockSpec((1, cg), lambda b, n: (0, 0)),
        ],
        out_specs=(pl.BlockSpec((1, tn, c64), lambda b, n: (b, n, 0)),
                   pl.BlockSpec((1, 1, cg), lambda b, n: (b, 0, 0))),
        compiler_params=pltpu.CompilerParams(
            dimension_semantics=("parallel", "arbitrary")),
    )(xp, w1p, _row(b1), w2, _row(b2), u1, _row(c1), u2, _row(c2), u3, _row(c3))
    return h3, pool.reshape(B, cg)


def _gfeat_body(h3_ref, w3_ref, b3_ref, w4_ref, b4_ref, w5_ref, b5_ref,
                mx_ref, ix_ref, *, tn, nsteps):
    nt = pl.program_id(1)

    @pl.when(nt == 0)
    def _():
        mx_ref[...] = jnp.full(mx_ref.shape, -jnp.inf, jnp.float32)
        ix_ref[...] = jnp.zeros(ix_ref.shape, jnp.int32)

    h = jnp.maximum(_dot(h3_ref[0], w3_ref[0]) + b3_ref[...], 0.0)
    h = jnp.maximum(_dot(h, w4_ref[...]) + b4_ref[...], 0.0)
    tmaxs, tidxs = [], []
    for c in range(0, w5_ref.shape[1], _CW):
        z = _dot(h, w5_ref[:, c:c + _CW])
        tmaxs.append(jnp.max(z, axis=0, keepdims=True))
        tidxs.append(jnp.argmax(z, axis=0).astype(jnp.int32)[None, :])
    tmax = jnp.concatenate(tmaxs, axis=1)
    tidx = jnp.concatenate(tidxs, axis=1) + nt * tn
    better = tmax > mx_ref[0]
    ix_ref[0] = jnp.where(better, tidx, ix_ref[0])
    mx_ref[0] = jnp.where(better, tmax, mx_ref[0])

    @pl.when(nt == nsteps - 1)
    def _():
        mz = mx_ref[0] + b5_ref[...]
        ix_ref[0] = jnp.where(mz > 0.0, ix_ref[0], 0)
        mx_ref[0] = jnp.maximum(mz, 0.0)


def _gfeat(h3, tn, w3p, b3, w4, b4, w5, b5):
    B, N, cin = h3.shape
    cg = w5.shape[1]
    mx, ix = pl.pallas_call(
        functools.partial(_gfeat_body, tn=tn, nsteps=N // tn),
        out_shape=(jax.ShapeDtypeStruct((B, 1, cg), jnp.float32),
                   jax.ShapeDtypeStruct((B, 1, cg), jnp.int32)),
        grid=(B, N // tn),
        in_specs=[
            pl.BlockSpec((1, tn, cin), lambda b, n: (b, n, 0)),
            pl.BlockSpec((1, cin, w3p.shape[2]), lambda b, n: (b, 0, 0)),
            pl.BlockSpec((1, w3p.shape[2]), lambda b, n: (0, 0)),
            pl.BlockSpec(w4.shape, lambda b, n: (0, 0)),
            pl.BlockSpec((1, w4.shape[1]), lambda b, n: (0, 0)),
            pl.BlockSpec(w5.shape, lambda b, n: (0, 0)),
            pl.BlockSpec((1, cg), lambda b, n: (0, 0)),
        ],
        out_specs=(pl.BlockSpec((1, 1, cg), lambda b, n: (b, 0, 0)),
                   pl.BlockSpec((1, 1, cg), lambda b, n: (b, 0, 0))),
        compiler_params=pltpu.CompilerParams(
            dimension_semantics=("parallel", "arbitrary")),
    )(h3, w3p, _row(b3), w4, _row(b4), w5, _row(b5))
    return mx.reshape(B, cg), ix.reshape(B, cg)


def _seg_body(h3_ref, wl_ref, gb_ref, w2_ref, b2_ref, w3_ref, b3_ref,
              w4_ref, b4_ref, o_ref):
    h = jnp.maximum(_dot(h3_ref[0], wl_ref[0]) + gb_ref[0], 0.0)
    h = jnp.maximum(_dot(h, w2_ref[...]) + b2_ref[...], 0.0)
    h = jnp.maximum(_dot(h, w3_ref[...]) + b3_ref[...], 0.0)
    m = w4_ref.shape[1]
    if m <= 4:
        w4 = w4_ref[...]
        cols = [jnp.sum(h * w4[:, c][None, :], axis=1, keepdims=True)
                for c in range(m)]
        o_ref[0] = (jnp.concatenate(cols, axis=1) + b4_ref[...]).T
    else:
        o_ref[0] = _dot(h, w4_ref[...]) + b4_ref[...]


def _seg(h3, tn, wlp, gbias, w2, b2, w3, b3, w4, b4):
    B, N, cin = h3.shape
    c1 = wlp.shape[2]
    m = w4.shape[1]
    return pl.pallas_call(
        _seg_body,
        out_shape=jax.ShapeDtypeStruct((B, m, N), jnp.float32),
        grid=(B, N // tn),
        in_specs=[
            pl.BlockSpec((1, tn, cin), lambda b, n: (b, n, 0)),
            pl.BlockSpec((1, cin, c1), lambda b, n: (b, 0, 0)),
            pl.BlockSpec((1, 1, c1), lambda b, n: (b, 0, 0)),
            pl.BlockSpec(w2.shape, lambda b, n: (0, 0)),
            pl.BlockSpec((1, w2.shape[1]), lambda b, n: (0, 0)),
            pl.BlockSpec(w3.shape, lambda b, n: (0, 0)),
            pl.BlockSpec((1, w3.shape[1]), lambda b, n: (0, 0)),
            pl.BlockSpec(w4.shape, lambda b, n: (0, 0)),
            pl.BlockSpec((1, m), lambda b, n: (0, 0)),
        ],
        out_specs=pl.BlockSpec((1, m, tn), lambda b, n: (b, 0, n)),
        compiler_params=pltpu.CompilerParams(
            dimension_semantics=("parallel", "arbitrary")),
    )(h3, wlp, gbias.reshape(B, 1, c1), w2, _row(b2), w3, _row(b3),
      w4, _row(b4))


def _tnet_fc(pooled, l1w, l1b, l2w, l2b, l3w, l3b, d):
    h = jnp.maximum(pooled @ l1w + l1b, 0.0)
    h = jnp.maximum(h @ l2w + l2b, 0.0)
    t = h @ l3w + l3b
    return t.reshape(-1, d, d) + jnp.eye(d, dtype=jnp.float32)


def kernel(t1_conv1_w, t1_conv1_b, t1_conv2_w, t1_conv2_b, t1_conv3_w, t1_conv3_b, t1_lin1_w, t1_lin1_b, t1_lin2_w, t1_lin2_b, t1_lin3_w, t1_lin3_b, t2_conv1_w, t2_conv1_b, t2_conv2_w, t2_conv2_b, t2_conv3_w, t2_conv3_b, t2_lin1_w, t2_lin1_b, t2_lin2_w, t2_lin2_b, t2_lin3_w, t2_lin3_b, bb_conv1_w, bb_conv1_b, bb_conv2_w, bb_conv2_b, bb_conv3_w, bb_conv3_b, bb_conv4_w, bb_conv4_b, bb_conv5_w, bb_conv5_b, seg_conv1_local, seg_conv1_global, seg_conv1_bias, seg_conv2_w, seg_conv2_b, seg_conv3_w, seg_conv3_b, seg_conv4_w, seg_conv4_b, x):
    xp = x.astype(jnp.float32)
    B, dim, N = xp.shape
    tn = _tile_n(N, 20480)
    tn4 = _tile_n(N, 10240)

    p1 = _mlp_pool(xp, tn, (t1_conv1_w, t1_conv1_b, t1_conv2_w, t1_conv2_b,
                            t1_conv3_w, t1_conv3_b))
    A_in = _tnet_fc(p1, t1_lin1_w, t1_lin1_b, t1_lin2_w, t1_lin2_b,
                    t1_lin3_w, t1_lin3_b, dim)

    w1p = jnp.einsum("bij,jk->bik", A_in, bb_conv1_w)
    h3, p2 = _backbone(xp, tn, w1p, bb_conv1_b, bb_conv2_w, bb_conv2_b,
                       t2_conv1_w, t2_conv1_b, t2_conv2_w, t2_conv2_b,
                       t2_conv3_w, t2_conv3_b)
    A_feat = _tnet_fc(p2, t2_lin1_w, t2_lin1_b, t2_lin2_w, t2_lin2_b,
                      t2_lin3_w, t2_lin3_b, 64)

    w3p = jnp.einsum("bij,jk->bik", A_feat, bb_conv3_w)
    g, crit_idx = _gfeat(h3, tn, w3p, bb_conv3_b, bb_conv4_w, bb_conv4_b,
                         bb_conv5_w, bb_conv5_b)

    wlp = jnp.einsum("bij,jk->bik", A_feat, seg_conv1_local)
    gbias = g @ seg_conv1_global + seg_conv1_bias
    logits = _seg(h3, tn4, wlp, gbias, seg_conv2_w, seg_conv2_b,
                  seg_conv3_w, seg_conv3_b, seg_conv4_w, seg_conv4_b)
    return jnp.transpose(logits, (0, 2, 1)), crit_idx, A_feat


# speedup vs baseline: 1.0898x; 1.0898x over previous
import functools

import jax
import jax.numpy as jnp
from jax.experimental import pallas as pl
from jax.experimental.pallas import tpu as pltpu


def _tile_n(n, cap):
    for t in (20480, 16384, 10240, 8192, 5120, 4096, 2048, 1024, 512, 256,
              128, 64, 32, 16, 8):
        if t <= cap and n % t == 0:
            return t
    return n


_CW = 256


def _pool_row(h, w_ref, b_ref):
    cout = w_ref.shape[1]
    rms = []
    for c in range(0, cout, _CW):
        z = _dot(h, w_ref[:, c:c + _CW])
        rms.append(jnp.max(z, axis=0, keepdims=True))
    return jnp.concatenate(rms, axis=1) + b_ref[...]


def _row(v):
    return v.reshape(1, -1)


def _dot(a, b):
    return jnp.dot(a, b, preferred_element_type=jnp.float32)


def _dot_ta(a, b):
    return jax.lax.dot_general(a, b, (((0,), (0,)), ((), ())),
                               preferred_element_type=jnp.float32)


def _mlp_pool_body(x_ref, *args):
    pool_ref = args[-1]

    @pl.when(pl.program_id(1) == 0)
    def _():
        pool_ref[...] = jnp.full(pool_ref.shape, -jnp.inf, jnp.float32)

    h = jnp.maximum(_dot_ta(x_ref[0], args[0][...]) + args[1][...], 0.0)
    for i in range(2, len(args) - 3, 2):
        h = jnp.maximum(_dot(h, args[i][...]) + args[i + 1][...], 0.0)
    rm = _pool_row(h, args[-3], args[-2])
    pool_ref[0] = jnp.maximum(pool_ref[0], jnp.maximum(rm, 0.0))


def _mlp_pool(xp, tn, wbs):
    B, cin, N = xp.shape
    cout = wbs[-2].shape[1]
    specs = [pl.BlockSpec((1, cin, tn), lambda b, n: (b, 0, n))]
    ops = []
    for w, b in zip(wbs[::2], wbs[1::2]):
        specs.append(pl.BlockSpec(w.shape, lambda b, n: (0, 0)))
        specs.append(pl.BlockSpec((1, w.shape[1]), lambda b, n: (0, 0)))
        ops += [w, _row(b)]
    out = pl.pallas_call(
        _mlp_pool_body,
        out_shape=jax.ShapeDtypeStruct((B, 1, cout), jnp.float32),
        grid=(B, N // tn),
        in_specs=specs,
        out_specs=pl.BlockSpec((1, 1, cout), lambda b, n: (b, 0, 0)),
        compiler_params=pltpu.CompilerParams(
            dimension_semantics=("parallel", "arbitrary")),
    )(xp, *ops)
    return out.reshape(B, cout)


def _backbone_body(x_ref, w1_ref, b1_ref, w2_ref, b2_ref,
                   u1_ref, c1_ref, u2_ref, c2_ref, u3_ref, c3_ref,
                   h3_ref, pool_ref):
    @pl.when(pl.program_id(1) == 0)
    def _():
        pool_ref[...] = jnp.full(pool_ref.shape, -jnp.inf, jnp.float32)

    h = jnp.maximum(_dot_ta(x_ref[0], w1_ref[0]) + b1_ref[...], 0.0)
    h3 = jnp.maximum(_dot(h, w2_ref[...]) + b2_ref[...], 0.0)
    h3_ref[0] = h3
    t = jnp.maximum(_dot(h3, u1_ref[...]) + c1_ref[...], 0.0)
    t = jnp.maximum(_dot(t, u2_ref[...]) + c2_ref[...], 0.0)
    rm = _pool_row(t, u3_ref, c3_ref)
    pool_ref[0] = jnp.maximum(pool_ref[0], jnp.maximum(rm, 0.0))


def _backbone(xp, tn, w1p, b1, w2, b2, u1, c1, u2, c2, u3, c3):
    B, cin, N = xp.shape
    c64 = w2.shape[1]
    cg = u3.shape[1]
    h3, pool = pl.pallas_call(
        _backbone_body,
        out_shape=(jax.ShapeDtypeStruct((B, N, c64), jnp.float32),
                   jax.ShapeDtypeStruct((B, 1, cg), jnp.float32)),
        grid=(B, N // tn),
        in_specs=[
            pl.BlockSpec((1, cin, tn), lambda b, n: (b, 0, n)),
            pl.BlockSpec((1, cin, w1p.shape[2]), lambda b, n: (b, 0, 0)),
            pl.BlockSpec((1, w1p.shape[2]), lambda b, n: (0, 0)),
            pl.BlockSpec(w2.shape, lambda b, n: (0, 0)),
            pl.BlockSpec((1, c64), lambda b, n: (0, 0)),
            pl.BlockSpec(u1.shape, lambda b, n: (0, 0)),
            pl.BlockSpec((1, u1.shape[1]), lambda b, n: (0, 0)),
            pl.BlockSpec(u2.shape, lambda b, n: (0, 0)),
            pl.BlockSpec((1, u2.shape[1]), lambda b, n: (0, 0)),
            pl.BlockSpec(u3.shape, lambda b, n: (0, 0)),
            pl.BlockSpec((1, cg), lambda b, n: (0, 0)),
        ],
        out_specs=(pl.BlockSpec((1, tn, c64), lambda b, n: (b, n, 0)),
                   pl.BlockSpec((1, 1, cg), lambda b, n: (b, 0, 0))),
        compiler_params=pltpu.CompilerParams(
            dimension_semantics=("parallel", "arbitrary")),
    )(xp, w1p, _row(b1), w2, _row(b2), u1, _row(c1), u2, _row(c2), u3, _row(c3))
    return h3, pool.reshape(B, cg)


def _gfeat_body(h3_ref, w3_ref, b3_ref, w4_ref, b4_ref, w5_ref, b5_ref,
                mx_ref, ix_ref, *, tn, nsteps):
    nt = pl.program_id(1)

    @pl.when(nt == 0)
    def _():
        mx_ref[...] = jnp.full(mx_ref.shape, -jnp.inf, jnp.float32)
        ix_ref[...] = jnp.zeros(ix_ref.shape, jnp.int32)

    h = jnp.maximum(_dot(h3_ref[0], w3_ref[0]) + b3_ref[...], 0.0)
    h = jnp.maximum(_dot(h, w4_ref[...]) + b4_ref[...], 0.0)
    tmaxs, tidxs = [], []
    for c in range(0, w5_ref.shape[1], _CW):
        z = _dot(h, w5_ref[:, c:c + _CW])
        tmaxs.append(jnp.max(z, axis=0, keepdims=True))
        tidxs.append(jnp.argmax(z, axis=0).astype(jnp.int32)[None, :])
    tmax = jnp.concatenate(tmaxs, axis=1)
    tidx = jnp.concatenate(tidxs, axis=1) + nt * tn
    better = tmax > mx_ref[0]
    ix_ref[0] = jnp.where(better, tidx, ix_ref[0])
    mx_ref[0] = jnp.where(better, tmax, mx_ref[0])

    @pl.when(nt == nsteps - 1)
    def _():
        mz = mx_ref[0] + b5_ref[...]
        ix_ref[0] = jnp.where(mz > 0.0, ix_ref[0], 0)
        mx_ref[0] = jnp.maximum(mz, 0.0)


def _gfeat(h3, tn, w3p, b3, w4, b4, w5, b5):
    B, N, cin = h3.shape
    cg = w5.shape[1]
    mx, ix = pl.pallas_call(
        functools.partial(_gfeat_body, tn=tn, nsteps=N // tn),
        out_shape=(jax.ShapeDtypeStruct((B, 1, cg), jnp.float32),
                   jax.ShapeDtypeStruct((B, 1, cg), jnp.int32)),
        grid=(B, N // tn),
        in_specs=[
            pl.BlockSpec((1, tn, cin), lambda b, n: (b, n, 0)),
            pl.BlockSpec((1, cin, w3p.shape[2]), lambda b, n: (b, 0, 0)),
            pl.BlockSpec((1, w3p.shape[2]), lambda b, n: (0, 0)),
            pl.BlockSpec(w4.shape, lambda b, n: (0, 0)),
            pl.BlockSpec((1, w4.shape[1]), lambda b, n: (0, 0)),
            pl.BlockSpec(w5.shape, lambda b, n: (0, 0)),
            pl.BlockSpec((1, cg), lambda b, n: (0, 0)),
        ],
        out_specs=(pl.BlockSpec((1, 1, cg), lambda b, n: (b, 0, 0)),
                   pl.BlockSpec((1, 1, cg), lambda b, n: (b, 0, 0))),
        compiler_params=pltpu.CompilerParams(
            dimension_semantics=("parallel", "arbitrary")),
    )(h3, w3p, _row(b3), w4, _row(b4), w5, _row(b5))
    return mx.reshape(B, cg), ix.reshape(B, cg)


def _seg_body(h3_ref, wl_ref, gb_ref, w2_ref, b2_ref, w3_ref, b3_ref,
              w4_ref, b4_ref, o_ref):
    h = jnp.maximum(_dot(h3_ref[0], wl_ref[0]) + gb_ref[0], 0.0)
    h = jnp.maximum(_dot(h, w2_ref[...]) + b2_ref[...], 0.0)
    h = jnp.maximum(_dot(h, w3_ref[...]) + b3_ref[...], 0.0)
    m = w4_ref.shape[1]
    if m <= 4:
        w4 = w4_ref[...]
        ht = h.T
        rows = [jnp.sum(ht * w4[:, c][:, None], axis=0, keepdims=True)
                for c in range(m)]
        o_ref[0] = jnp.concatenate(rows, axis=0)
    else:
        o_ref[0] = _dot(h, w4_ref[...]) + b4_ref[...]


def _seg(h3, tn, wlp, gbias, w2, b2, w3, b3, w4, b4):
    B, N, cin = h3.shape
    c1 = wlp.shape[2]
    m = w4.shape[1]
    return pl.pallas_call(
        _seg_body,
        out_shape=jax.ShapeDtypeStruct((B, m, N), jnp.float32),
        grid=(B, N // tn),
        in_specs=[
            pl.BlockSpec((1, tn, cin), lambda b, n: (b, n, 0)),
            pl.BlockSpec((1, cin, c1), lambda b, n: (b, 0, 0)),
            pl.BlockSpec((1, 1, c1), lambda b, n: (b, 0, 0)),
            pl.BlockSpec(w2.shape, lambda b, n: (0, 0)),
            pl.BlockSpec((1, w2.shape[1]), lambda b, n: (0, 0)),
            pl.BlockSpec(w3.shape, lambda b, n: (0, 0)),
            pl.BlockSpec((1, w3.shape[1]), lambda b, n: (0, 0)),
            pl.BlockSpec(w4.shape, lambda b, n: (0, 0)),
            pl.BlockSpec((1, m), lambda b, n: (0, 0)),
        ],
        out_specs=pl.BlockSpec((1, m, tn), lambda b, n: (b, 0, n)),
        compiler_params=pltpu.CompilerParams(
            dimension_semantics=("parallel", "arbitrary")),
    )(h3, wlp, gbias.reshape(B, 1, c1), w2, _row(b2), w3, _row(b3),
      w4, _row(b4))


def _tnet_fc(pooled, l1w, l1b, l2w, l2b, l3w, l3b, d):
    h = jnp.maximum(pooled @ l1w + l1b, 0.0)
    h = jnp.maximum(h @ l2w + l2b, 0.0)
    t = h @ l3w + l3b
    return t.reshape(-1, d, d) + jnp.eye(d, dtype=jnp.float32)


def kernel(t1_conv1_w, t1_conv1_b, t1_conv2_w, t1_conv2_b, t1_conv3_w, t1_conv3_b, t1_lin1_w, t1_lin1_b, t1_lin2_w, t1_lin2_b, t1_lin3_w, t1_lin3_b, t2_conv1_w, t2_conv1_b, t2_conv2_w, t2_conv2_b, t2_conv3_w, t2_conv3_b, t2_lin1_w, t2_lin1_b, t2_lin2_w, t2_lin2_b, t2_lin3_w, t2_lin3_b, bb_conv1_w, bb_conv1_b, bb_conv2_w, bb_conv2_b, bb_conv3_w, bb_conv3_b, bb_conv4_w, bb_conv4_b, bb_conv5_w, bb_conv5_b, seg_conv1_local, seg_conv1_global, seg_conv1_bias, seg_conv2_w, seg_conv2_b, seg_conv3_w, seg_conv3_b, seg_conv4_w, seg_conv4_b, x):
    xp = x.astype(jnp.float32)
    B, dim, N = xp.shape
    tn = _tile_n(N, 20480)
    tn4 = _tile_n(N, 10240)

    p1 = _mlp_pool(xp, tn, (t1_conv1_w, t1_conv1_b, t1_conv2_w, t1_conv2_b,
                            t1_conv3_w, t1_conv3_b))
    A_in = _tnet_fc(p1, t1_lin1_w, t1_lin1_b, t1_lin2_w, t1_lin2_b,
                    t1_lin3_w, t1_lin3_b, dim)

    w1p = jnp.einsum("bij,jk->bik", A_in, bb_conv1_w)
    h3, p2 = _backbone(xp, tn, w1p, bb_conv1_b, bb_conv2_w, bb_conv2_b,
                       t2_conv1_w, t2_conv1_b, t2_conv2_w, t2_conv2_b,
                       t2_conv3_w, t2_conv3_b)
    A_feat = _tnet_fc(p2, t2_lin1_w, t2_lin1_b, t2_lin2_w, t2_lin2_b,
                      t2_lin3_w, t2_lin3_b, 64)

    w3p = jnp.einsum("bij,jk->bik", A_feat, bb_conv3_w)
    g, crit_idx = _gfeat(h3, tn, w3p, bb_conv3_b, bb_conv4_w, bb_conv4_b,
                         bb_conv5_w, bb_conv5_b)

    wlp = jnp.einsum("bij,jk->bik", A_feat, seg_conv1_local)
    gbias = g @ seg_conv1_global + seg_conv1_bias
    logits_t = _seg(h3, tn4, wlp, gbias, seg_conv2_w, seg_conv2_b,
                    seg_conv3_w, seg_conv3_b, seg_conv4_w, seg_conv4_b)
    logits = jnp.transpose(logits_t, (0, 2, 1)) + seg_conv4_b[None, None, :]
    return logits, crit_idx, A_feat
